# baseline (speedup 1.0000x reference)
import functools

import jax
import jax.numpy as jnp
from jax import lax
from jax.experimental import pallas as pl
from jax.experimental.pallas import tpu as pltpu
from jax.experimental.pallas import tpu_sc as plsc

C = 21
B = 32
P = 24576
O = 20
ROWS = 96
LANES = 256
SCH = 12
TCH = 2048


def _iota2(shape):
    ri = lax.broadcasted_iota(jnp.int32, shape, 0)
    li = lax.broadcasted_iota(jnp.int32, shape, 1)
    return ri, li


def _match_kernel(pt_ref, tgt_ref, bto_ref, bti_ref, bpi_ref):
    pt = pt_ref[...]
    pcx, pcy, pw, ph = pt[0], pt[1], pt[2], pt[3]
    px1 = pcx - pw * 0.5
    py1 = pcy - ph * 0.5
    px2 = pcx + pw * 0.5
    py2 = pcy + ph * 0.5
    pa = (px2 - px1) * (py2 - py1)
    ri, li = _iota2((ROWS, LANES))
    gidx = ri * LANES + li

    b_ov = jnp.full((ROWS, LANES), -1.0, jnp.float32)
    b_ix = jnp.zeros((ROWS, LANES), jnp.int32)
    rm_list, cm_list = [], []
    for j in range(O):
        tx1 = tgt_ref[0, j, 0]
        ty1 = tgt_ref[0, j, 1]
        tx2 = tgt_ref[0, j, 2]
        ty2 = tgt_ref[0, j, 3]
        ta = (tx2 - tx1) * (ty2 - ty1)
        ix = jnp.maximum(jnp.minimum(tx2, px2) - jnp.maximum(tx1, px1), 0.0)
        iy = jnp.maximum(jnp.minimum(ty2, py2) - jnp.maximum(ty1, py1), 0.0)
        inter = ix * iy
        ov = inter / (ta + pa - inter)
        upd = ov > b_ov
        b_ov = jnp.where(upd, ov, b_ov)
        b_ix = jnp.where(upd, j, b_ix)
        rm = jnp.max(ov, axis=0)
        cm = jnp.min(jnp.where(ov == rm[None, :], gidx, P), axis=0)
        rm_list.append(rm)
        cm_list.append(cm)

    RM = jnp.stack(rm_list)
    CM = jnp.stack(cm_list)
    M = jnp.max(RM, axis=1, keepdims=True)
    idxv = jnp.min(jnp.where(RM == M, CM, P), axis=1)
    bpi_ref[...] = jnp.concatenate(
        [idxv, jnp.full((32 - O,), P, jnp.int32)]).reshape(1, 1, 32)

    bto_ref[0] = b_ov
    bti_ref[0] = b_ix


def _trans_kernel(preds_ref, confT_ref, locT_ref):
    xt = jnp.swapaxes(preds_ref[0], 0, 1)
    confT_ref[0] = xt[4:25].reshape(C, TCH // LANES, LANES)
    locT_ref[0] = xt[0:4].reshape(4, TCH // LANES, LANES)


def _tc_trans(predictions):
    return pl.pallas_call(
        _trans_kernel,
        grid=(B, SCH),
        in_specs=[pl.BlockSpec((1, TCH, 25), lambda b, s: (b, s, 0))],
        out_specs=[
            pl.BlockSpec((1, C, TCH // LANES, LANES), lambda b, s: (b, 0, s, 0)),
            pl.BlockSpec((1, 4, TCH // LANES, LANES), lambda b, s: (b, 0, s, 0)),
        ],
        out_shape=[
            jax.ShapeDtypeStruct((B, C, ROWS, LANES), jnp.float32),
            jax.ShapeDtypeStruct((B, 4, ROWS, LANES), jnp.float32),
        ],
        compiler_params=pltpu.CompilerParams(
            dimension_semantics=("arbitrary", "arbitrary")),
    )(predictions)


def _loss_kernel(confT_ref, locT_ref, pt_ref, tgt_ref, bto_ref, bti_ref,
                 bpi_ref, ce_ref, mval_ref, scal_ref):
    ri, li = _iota2((ROWS, LANES))
    gidx = ri * LANES + li

    bto = bto_ref[0]
    bti = bti_ref[0]
    sl32 = lax.broadcasted_iota(jnp.int32, (1, 1, 32), 2)
    bv = bpi_ref[...]
    for j in range(O):
        mfm = gidx == jnp.sum(jnp.where(sl32 == j, bv, 0))
        bto = jnp.where(mfm, 2.0, bto)
        bti = jnp.where(mfm, j, bti)
    pos = bto >= 0.5
    posf = pos.astype(jnp.float32)

    lab = jnp.zeros((ROWS, LANES), jnp.float32)
    mx1 = jnp.zeros((ROWS, LANES), jnp.float32)
    my1 = jnp.zeros((ROWS, LANES), jnp.float32)
    mx2 = jnp.zeros((ROWS, LANES), jnp.float32)
    my2 = jnp.zeros((ROWS, LANES), jnp.float32)
    for j in range(O):
        mj = bti == j
        lab = jnp.where(mj, tgt_ref[0, j, 4], lab)
        mx1 = jnp.where(mj, tgt_ref[0, j, 0], mx1)
        my1 = jnp.where(mj, tgt_ref[0, j, 1], my1)
        mx2 = jnp.where(mj, tgt_ref[0, j, 2], mx2)
        my2 = jnp.where(mj, tgt_ref[0, j, 3], my2)
    cj = jnp.where(pos, lab.astype(jnp.int32) + 1, 0)

    x = confT_ref[0]
    m = jnp.max(x, axis=0)
    e = jnp.exp(x - m[None])
    ssum = jnp.sum(e, axis=0)
    lse = jnp.log(ssum) + m
    ci = lax.broadcasted_iota(jnp.int32, (C, ROWS, LANES), 0)
    tgt = jnp.sum(jnp.where(ci == cj[None], x, 0.0), axis=0)
    ce = lse - tgt
    mval = jnp.where(pos, 0.0, ce)
    ce_ref[0] = ce
    mval_ref[0] = mval

    pt = pt_ref[...]
    pcx, pcy, pw, ph = pt[0], pt[1], pt[2], pt[3]
    g_cx = ((mx1 + mx2) * 0.5 - pcx) / (0.1 * pw)
    g_cy = ((my1 + my2) * 0.5 - pcy) / (0.1 * ph)
    g_w = jnp.log(jnp.maximum((mx2 - mx1) / pw, 1e-8)) / jnp.float32(0.2)
    g_h = jnp.log(jnp.maximum((my2 - my1) / ph, 1e-8)) / jnp.float32(0.2)
    lp = locT_ref[0]

    def sl1(d):
        ad = jnp.abs(d)
        return jnp.where(ad < 1.0, 0.5 * d * d, ad - 0.5)

    lsum = jnp.sum((sl1(lp[0] - g_cx) + sl1(lp[1] - g_cy)
                    + sl1(lp[2] - g_w) + sl1(lp[3] - g_h)) * posf)
    npc = jnp.sum(posf)

    slc = lax.broadcasted_iota(jnp.int32, (1, 1, 128), 2)
    scal_ref[...] = jnp.where(slc == 0, npc, 0.0) + jnp.where(slc == 1, lsum, 0.0)


def _tc_match(pt, targets):
    return pl.pallas_call(
        _match_kernel,
        grid=(B,),
        in_specs=[
            pl.BlockSpec((4, ROWS, LANES), lambda b: (0, 0, 0)),
            pl.BlockSpec((1, O, 5), lambda b: (b, 0, 0)),
        ],
        out_specs=[
            pl.BlockSpec((1, ROWS, LANES), lambda b: (b, 0, 0)),
            pl.BlockSpec((1, ROWS, LANES), lambda b: (b, 0, 0)),
            pl.BlockSpec((1, 1, 32), lambda b: (b, 0, 0)),
        ],
        out_shape=[
            jax.ShapeDtypeStruct((B, ROWS, LANES), jnp.float32),
            jax.ShapeDtypeStruct((B, ROWS, LANES), jnp.int32),
            jax.ShapeDtypeStruct((B, 1, 32), jnp.int32),
        ],
        compiler_params=pltpu.CompilerParams(
            dimension_semantics=("arbitrary",)),
    )(pt, targets)


def _tc_loss(confT, locT, pt, targets, bto, bti, bpi):
    return pl.pallas_call(
        _loss_kernel,
        grid=(B,),
        in_specs=[
            pl.BlockSpec((1, C, ROWS, LANES), lambda b: (b, 0, 0, 0)),
            pl.BlockSpec((1, 4, ROWS, LANES), lambda b: (b, 0, 0, 0)),
            pl.BlockSpec((4, ROWS, LANES), lambda b: (0, 0, 0)),
            pl.BlockSpec((1, O, 5), lambda b: (b, 0, 0)),
            pl.BlockSpec((1, ROWS, LANES), lambda b: (b, 0, 0)),
            pl.BlockSpec((1, ROWS, LANES), lambda b: (b, 0, 0)),
            pl.BlockSpec((1, 1, 32), lambda b: (b, 0, 0)),
        ],
        out_specs=[
            pl.BlockSpec((1, ROWS, LANES), lambda b: (b, 0, 0)),
            pl.BlockSpec((1, ROWS, LANES), lambda b: (b, 0, 0)),
            pl.BlockSpec((1, 1, 128), lambda b: (b, 0, 0)),
        ],
        out_shape=[
            jax.ShapeDtypeStruct((B, ROWS, LANES), jnp.float32),
            jax.ShapeDtypeStruct((B, ROWS, LANES), jnp.float32),
            jax.ShapeDtypeStruct((B, 1, 128), jnp.float32),
        ],
        compiler_params=pltpu.CompilerParams(
            dimension_semantics=("arbitrary",)),
    )(confT, locT, pt, targets, bto, bti, bpi)


NV = P // 16


def _take16(x, idx):
    return x.at[idx].get(mode="promise_in_bounds")


def _lanesum(x):
    i16 = lax.iota(jnp.int32, 16)
    for d in (8, 4, 2, 1):
        x = x + _take16(x, i16 ^ d)
    return x


def _lanemin(x):
    i16 = lax.iota(jnp.int32, 16)
    for d in (8, 4, 2, 1):
        x = jnp.minimum(x, _take16(x, i16 ^ d))
    return x


def _cumsum16(x):
    i16 = lax.iota(jnp.int32, 16)
    for d in (1, 2, 4, 8):
        sh = _take16(x, jnp.maximum(i16 - d, 0))
        x = x + jnp.where(i16 >= d, sh, 0)
    return x


def _sc_mine_kernel(mval_hbm, ce_hbm, scal_hbm, out_hbm,
                    mv_ref, ce_ref, sc_ref, acc_ref, sti_ref, stf_ref):
    wid = lax.axis_index("s") * 2 + lax.axis_index("c")
    pltpu.sync_copy(mval_hbm.at[wid], mv_ref)
    pltpu.sync_copy(ce_hbm.at[wid], ce_ref)
    pltpu.sync_copy(scal_hbm.at[wid], sc_ref)

    i16 = lax.iota(jnp.int32, 16)
    zeros16i = jnp.zeros((16,), jnp.int32)

    head = sc_ref[pl.ds(0, 16)].astype(jnp.int32)
    npos = _lanesum(jnp.where(i16 == 0, head, 0))
    k = jnp.minimum(3 * npos, P - 1)

    inf16 = jnp.full((16,), jnp.inf, jnp.float32)
    sti_ref[pl.ds(0, 16)] = zeros16i
    sti_ref[pl.ds(16, 16)] = zeros16i
    stf_ref[...] = inf16

    def round_body(r, carry):
        def active(_):
            prefix = sti_ref[pl.ds(0, 16)]
            trial = prefix | lax.shift_left(jnp.int32(1), 31 - r)
            trialf = lax.bitcast_convert_type(trial, jnp.float32)

            def cnt_body(i, cs):
                c0, c1, c2, c3, sm = cs
                b = i * 64
                v0 = mv_ref[pl.ds(b, 16)]
                v1 = mv_ref[pl.ds(b + 16, 16)]
                v2 = mv_ref[pl.ds(b + 32, 16)]
                v3 = mv_ref[pl.ds(b + 48, 16)]
                c0 = c0 + jnp.where(v0 >= trialf, 1, 0)
                c1 = c1 + jnp.where(v1 >= trialf, 1, 0)
                c2 = c2 + jnp.where(v2 >= trialf, 1, 0)
                c3 = c3 + jnp.where(v3 >= trialf, 1, 0)
                m01 = jnp.minimum(jnp.where(v0 >= trialf, v0, jnp.inf),
                                  jnp.where(v1 >= trialf, v1, jnp.inf))
                m23 = jnp.minimum(jnp.where(v2 >= trialf, v2, jnp.inf),
                                  jnp.where(v3 >= trialf, v3, jnp.inf))
                return (c0, c1, c2, c3, jnp.minimum(sm, jnp.minimum(m01, m23)))

            cl = lax.fori_loop(0, NV // 4, cnt_body,
                               (zeros16i, zeros16i, zeros16i, zeros16i, inf16))
            cnt = _lanesum(cl[0] + cl[1] + cl[2] + cl[3])
            hit = cnt == k
            sti_ref[pl.ds(0, 16)] = jnp.where(cnt >= k, trial, prefix)
            sti_ref[pl.ds(16, 16)] = jnp.where(hit, 1, 0)
            stf_ref[...] = jnp.where(hit, _lanemin(cl[4]), inf16)
            return jnp.int32(0)

        done = sti_ref[pl.ds(16, 16)]
        return lax.cond(done[0] == 1, lambda _: jnp.int32(0), active, 0)

    lax.fori_loop(1, 32, round_body, jnp.int32(0))
    done = sti_ref[pl.ds(16, 16)]
    tvec = jnp.where(done == 1, stf_ref[...],
                     lax.bitcast_convert_type(sti_ref[pl.ds(0, 16)], jnp.float32))

    def gt_body(i, c):
        mv = mv_ref[pl.ds(i * 16, 16)]
        return c + jnp.where(mv > tvec, 1, 0)

    cnt_gt = _lanesum(lax.fori_loop(0, NV, gt_body, zeros16i))
    m_extra = jnp.where(k > 0, k - cnt_gt, 0)
    tvec = jnp.where(k > 0, tvec, jnp.float32(jnp.inf))

    def fin_body(i, carry):
        cnt_eq, acc = carry
        mv = mv_ref[pl.ds(i * 16, 16)]
        cev = ce_ref[pl.ds(i * 16, 16)]
        eq = mv == tvec
        eqi = jnp.where(eq, 1, 0)
        c = _cumsum16(eqi)
        sel = eq & ((cnt_eq + c) <= m_extra)
        mask = (mv == 0.0) | (mv > tvec) | sel
        acc = acc + jnp.where(mask, cev, 0.0)
        return cnt_eq + _lanesum(eqi), acc

    _, acc = lax.fori_loop(0, NV, fin_body,
                           (zeros16i, jnp.zeros((16,), jnp.float32)))
    acc_ref[...] = acc
    pltpu.sync_copy(acc_ref, out_hbm.at[wid])


def _sc_mine(mval, ce, scal):
    mesh = plsc.VectorSubcoreMesh(core_axis_name="c", subcore_axis_name="s")
    f = pl.kernel(
        _sc_mine_kernel,
        mesh=mesh,
        out_type=jax.ShapeDtypeStruct((B, 16), jnp.float32),
        scratch_types=[
            pltpu.VMEM((P,), jnp.float32),
            pltpu.VMEM((P,), jnp.float32),
            pltpu.VMEM((128,), jnp.float32),
            pltpu.VMEM((16,), jnp.float32),
            pltpu.VMEM((32,), jnp.int32),
            pltpu.VMEM((16,), jnp.float32),
        ],
    )
    return f(mval, ce, scal)


def _sel_one(mv, cev, kk):
    bits = lax.bitcast_convert_type(mv, jnp.int32)
    prefix = jnp.int32(0)
    kp = kk.astype(jnp.int32)
    shifts_c = [31, 24, 16, 8]
    shifts_d = [24, 16, 8, 0]
    for r in range(4):
        cand = lax.shift_right_logical(bits, shifts_c[r]) == prefix
        digit = lax.shift_right_logical(bits, shifts_d[r]) & 255
        hist = jnp.zeros((256,), jnp.int32).at[digit].add(cand.astype(jnp.int32))
        suf = jnp.cumsum(hist[::-1])[::-1]
        ok = suf >= kp
        dstar = jnp.max(jnp.where(ok, jnp.arange(256), -1))
        cnt_gt = suf[jnp.minimum(dstar + 1, 255)] * (dstar < 255)
        kp = kp - cnt_gt
        prefix = (prefix << 8) | dstar
    t = lax.bitcast_convert_type(prefix, jnp.float32)
    m_extra = kp
    t = jnp.where(kk > 0, t, jnp.float32(jnp.inf))
    m_extra = jnp.where(kk > 0, m_extra, 0)
    eq = mv == t
    csum_eq = jnp.cumsum(eq.astype(jnp.int32))
    sel_eq = eq & (csum_eq <= m_extra)
    mask = (mv == 0.0) | (mv > t) | sel_eq
    return jnp.sum(jnp.where(mask, cev, 0.0))


def kernel(predictions, targets, priors):
    pt = priors.T.reshape(4, ROWS, LANES)
    confT = jnp.transpose(predictions[:, :, 4:], (0, 2, 1)).reshape(B, C, ROWS, LANES)
    locT = jnp.transpose(predictions[:, :, :4], (0, 2, 1)).reshape(B, 4, ROWS, LANES)

    bto, bti, bpi = _tc_match(pt, targets)
    ce, mval, scal = _tc_loss(confT, locT, pt, targets, bto, bti, bpi)

    npos = scal[:, 0, 0]
    csum = _sc_mine(mval.reshape(B, P), ce.reshape(B, P), scal.reshape(B, 128))

    bp = jnp.sum(npos)
    loss_l = jnp.sum(scal[:, 0, 1]) / bp
    loss_c = jnp.sum(csum) / bp
    return loss_l, loss_c


# speedup vs baseline: 1.0200x; 1.0200x over previous
import functools

import jax
import jax.numpy as jnp
from jax import lax
from jax.experimental import pallas as pl
from jax.experimental.pallas import tpu as pltpu
from jax.experimental.pallas import tpu_sc as plsc

C = 21
B = 32
P = 24576
O = 20
ROWS = 96
LANES = 256
SCH = 12
TCH = 2048


def _iota2(shape):
    ri = lax.broadcasted_iota(jnp.int32, shape, 0)
    li = lax.broadcasted_iota(jnp.int32, shape, 1)
    return ri, li


def _match_kernel(pt_ref, tgt_ref, bto_ref, bti_ref, bpi_ref):
    pt = pt_ref[...]
    pcx, pcy, pw, ph = pt[0], pt[1], pt[2], pt[3]
    px1 = pcx - pw * 0.5
    py1 = pcy - ph * 0.5
    px2 = pcx + pw * 0.5
    py2 = pcy + ph * 0.5
    pa = (px2 - px1) * (py2 - py1)
    ri, li = _iota2((ROWS, LANES))
    gidx = ri * LANES + li

    b_ov = jnp.full((ROWS, LANES), -1.0, jnp.float32)
    b_ix = jnp.zeros((ROWS, LANES), jnp.int32)
    rm_list, cm_list = [], []
    for j in range(O):
        tx1 = tgt_ref[0, j, 0]
        ty1 = tgt_ref[0, j, 1]
        tx2 = tgt_ref[0, j, 2]
        ty2 = tgt_ref[0, j, 3]
        ta = (tx2 - tx1) * (ty2 - ty1)
        ix = jnp.maximum(jnp.minimum(tx2, px2) - jnp.maximum(tx1, px1), 0.0)
        iy = jnp.maximum(jnp.minimum(ty2, py2) - jnp.maximum(ty1, py1), 0.0)
        inter = ix * iy
        ov = inter / (ta + pa - inter)
        upd = ov > b_ov
        b_ov = jnp.where(upd, ov, b_ov)
        b_ix = jnp.where(upd, j, b_ix)
        rm = jnp.max(ov, axis=0)
        cm = jnp.min(jnp.where(ov == rm[None, :], gidx, P), axis=0)
        rm_list.append(rm)
        cm_list.append(cm)

    RM = jnp.stack(rm_list)
    CM = jnp.stack(cm_list)
    M = jnp.max(RM, axis=1, keepdims=True)
    idxv = jnp.min(jnp.where(RM == M, CM, P), axis=1)
    bpi_ref[...] = jnp.concatenate(
        [idxv, jnp.full((32 - O,), P, jnp.int32)]).reshape(1, 1, 32)

    bto_ref[0] = b_ov
    bti_ref[0] = b_ix


def _trans_kernel(preds_ref, confT_ref, locT_ref):
    xt = jnp.swapaxes(preds_ref[0], 0, 1)
    confT_ref[0] = xt[4:25].reshape(C, TCH // LANES, LANES)
    locT_ref[0] = xt[0:4].reshape(4, TCH // LANES, LANES)


def _tc_trans(predictions):
    return pl.pallas_call(
        _trans_kernel,
        grid=(B, SCH),
        in_specs=[pl.BlockSpec((1, TCH, 25), lambda b, s: (b, s, 0))],
        out_specs=[
            pl.BlockSpec((1, C, TCH // LANES, LANES), lambda b, s: (b, 0, s, 0)),
            pl.BlockSpec((1, 4, TCH // LANES, LANES), lambda b, s: (b, 0, s, 0)),
        ],
        out_shape=[
            jax.ShapeDtypeStruct((B, C, ROWS, LANES), jnp.float32),
            jax.ShapeDtypeStruct((B, 4, ROWS, LANES), jnp.float32),
        ],
        compiler_params=pltpu.CompilerParams(
            dimension_semantics=("arbitrary", "arbitrary")),
    )(predictions)


def _loss_kernel(confT_ref, locT_ref, pt_ref, tgt_ref, bto_ref, bti_ref,
                 bpi_ref, ce_ref, mval_ref, scal_ref):
    ri, li = _iota2((ROWS, LANES))
    gidx = ri * LANES + li

    bto = bto_ref[0]
    bti = bti_ref[0]
    sl32 = lax.broadcasted_iota(jnp.int32, (1, 1, 32), 2)
    bv = bpi_ref[...]
    for j in range(O):
        mfm = gidx == jnp.sum(jnp.where(sl32 == j, bv, 0))
        bto = jnp.where(mfm, 2.0, bto)
        bti = jnp.where(mfm, j, bti)
    pos = bto >= 0.5
    posf = pos.astype(jnp.float32)

    lab = jnp.zeros((ROWS, LANES), jnp.float32)
    mx1 = jnp.zeros((ROWS, LANES), jnp.float32)
    my1 = jnp.zeros((ROWS, LANES), jnp.float32)
    mx2 = jnp.zeros((ROWS, LANES), jnp.float32)
    my2 = jnp.zeros((ROWS, LANES), jnp.float32)
    for j in range(O):
        mj = bti == j
        lab = jnp.where(mj, tgt_ref[0, j, 4], lab)
        mx1 = jnp.where(mj, tgt_ref[0, j, 0], mx1)
        my1 = jnp.where(mj, tgt_ref[0, j, 1], my1)
        mx2 = jnp.where(mj, tgt_ref[0, j, 2], mx2)
        my2 = jnp.where(mj, tgt_ref[0, j, 3], my2)
    cj = jnp.where(pos, lab.astype(jnp.int32) + 1, 0)

    x = confT_ref[0]
    m = jnp.max(x, axis=0)
    e = jnp.exp(x - m[None])
    ssum = jnp.sum(e, axis=0)
    lse = jnp.log(ssum) + m
    ci = lax.broadcasted_iota(jnp.int32, (C, ROWS, LANES), 0)
    tgt = jnp.sum(jnp.where(ci == cj[None], x, 0.0), axis=0)
    ce = lse - tgt
    mval = jnp.where(pos, 0.0, ce)
    ce_ref[0] = ce
    mval_ref[0] = mval

    pt = pt_ref[...]
    pcx, pcy, pw, ph = pt[0], pt[1], pt[2], pt[3]
    g_cx = ((mx1 + mx2) * 0.5 - pcx) / (0.1 * pw)
    g_cy = ((my1 + my2) * 0.5 - pcy) / (0.1 * ph)
    g_w = jnp.log(jnp.maximum((mx2 - mx1) / pw, 1e-8)) / jnp.float32(0.2)
    g_h = jnp.log(jnp.maximum((my2 - my1) / ph, 1e-8)) / jnp.float32(0.2)
    lp = locT_ref[0]

    def sl1(d):
        ad = jnp.abs(d)
        return jnp.where(ad < 1.0, 0.5 * d * d, ad - 0.5)

    lsum = jnp.sum((sl1(lp[0] - g_cx) + sl1(lp[1] - g_cy)
                    + sl1(lp[2] - g_w) + sl1(lp[3] - g_h)) * posf)
    npc = jnp.sum(posf)

    slc = lax.broadcasted_iota(jnp.int32, (1, 1, 128), 2)
    scal_ref[...] = jnp.where(slc == 0, npc, 0.0) + jnp.where(slc == 1, lsum, 0.0)


def _tc_match(pt, targets):
    return pl.pallas_call(
        _match_kernel,
        grid=(B,),
        in_specs=[
            pl.BlockSpec((4, ROWS, LANES), lambda b: (0, 0, 0)),
            pl.BlockSpec((1, O, 5), lambda b: (b, 0, 0)),
        ],
        out_specs=[
            pl.BlockSpec((1, ROWS, LANES), lambda b: (b, 0, 0)),
            pl.BlockSpec((1, ROWS, LANES), lambda b: (b, 0, 0)),
            pl.BlockSpec((1, 1, 32), lambda b: (b, 0, 0)),
        ],
        out_shape=[
            jax.ShapeDtypeStruct((B, ROWS, LANES), jnp.float32),
            jax.ShapeDtypeStruct((B, ROWS, LANES), jnp.int32),
            jax.ShapeDtypeStruct((B, 1, 32), jnp.int32),
        ],
        compiler_params=pltpu.CompilerParams(
            dimension_semantics=("arbitrary",)),
    )(pt, targets)


def _tc_loss(confT, locT, pt, targets, bto, bti, bpi):
    return pl.pallas_call(
        _loss_kernel,
        grid=(B,),
        in_specs=[
            pl.BlockSpec((1, C, ROWS, LANES), lambda b: (b, 0, 0, 0)),
            pl.BlockSpec((1, 4, ROWS, LANES), lambda b: (b, 0, 0, 0)),
            pl.BlockSpec((4, ROWS, LANES), lambda b: (0, 0, 0)),
            pl.BlockSpec((1, O, 5), lambda b: (b, 0, 0)),
            pl.BlockSpec((1, ROWS, LANES), lambda b: (b, 0, 0)),
            pl.BlockSpec((1, ROWS, LANES), lambda b: (b, 0, 0)),
            pl.BlockSpec((1, 1, 32), lambda b: (b, 0, 0)),
        ],
        out_specs=[
            pl.BlockSpec((1, ROWS, LANES), lambda b: (b, 0, 0)),
            pl.BlockSpec((1, ROWS, LANES), lambda b: (b, 0, 0)),
            pl.BlockSpec((1, 1, 128), lambda b: (b, 0, 0)),
        ],
        out_shape=[
            jax.ShapeDtypeStruct((B, ROWS, LANES), jnp.float32),
            jax.ShapeDtypeStruct((B, ROWS, LANES), jnp.float32),
            jax.ShapeDtypeStruct((B, 1, 128), jnp.float32),
        ],
        compiler_params=pltpu.CompilerParams(
            dimension_semantics=("arbitrary",)),
    )(confT, locT, pt, targets, bto, bti, bpi)


NV = P // 16


def _take16(x, idx):
    return x.at[idx].get(mode="promise_in_bounds")


def _lanesum(x):
    i16 = lax.iota(jnp.int32, 16)
    for d in (8, 4, 2, 1):
        x = x + _take16(x, i16 ^ d)
    return x


def _lanemin(x):
    i16 = lax.iota(jnp.int32, 16)
    for d in (8, 4, 2, 1):
        x = jnp.minimum(x, _take16(x, i16 ^ d))
    return x


def _cumsum16(x):
    i16 = lax.iota(jnp.int32, 16)
    for d in (1, 2, 4, 8):
        sh = _take16(x, jnp.maximum(i16 - d, 0))
        x = x + jnp.where(i16 >= d, sh, 0)
    return x


def _sc_mine_kernel(mval_hbm, ce_hbm, scal_hbm, out_hbm,
                    mv_ref, ce_ref, sc_ref, acc_ref, sti_ref, stf_ref):
    wid = lax.axis_index("s") * 2 + lax.axis_index("c")
    pltpu.sync_copy(mval_hbm.at[wid], mv_ref)
    pltpu.sync_copy(ce_hbm.at[wid], ce_ref)
    pltpu.sync_copy(scal_hbm.at[wid], sc_ref)

    i16 = lax.iota(jnp.int32, 16)
    zeros16i = jnp.zeros((16,), jnp.int32)

    head = sc_ref[pl.ds(0, 16)].astype(jnp.int32)
    npos = _lanesum(jnp.where(i16 == 0, head, 0))
    k = jnp.minimum(3 * npos, P - 1)

    inf16 = jnp.full((16,), jnp.inf, jnp.float32)
    sti_ref[pl.ds(0, 16)] = zeros16i
    sti_ref[pl.ds(16, 16)] = zeros16i
    stf_ref[...] = inf16

    def round_body(r, carry):
        def active(_):
            prefix = sti_ref[pl.ds(0, 16)]
            trial = prefix | lax.shift_left(jnp.int32(1), 31 - r)
            trialf = lax.bitcast_convert_type(trial, jnp.float32)

            def cnt_body(i, cs):
                c0, c1, c2, c3 = cs
                b = i * 64
                c0 = c0 + jnp.where(mv_ref[pl.ds(b, 16)] >= trialf, 1, 0)
                c1 = c1 + jnp.where(mv_ref[pl.ds(b + 16, 16)] >= trialf, 1, 0)
                c2 = c2 + jnp.where(mv_ref[pl.ds(b + 32, 16)] >= trialf, 1, 0)
                c3 = c3 + jnp.where(mv_ref[pl.ds(b + 48, 16)] >= trialf, 1, 0)
                return (c0, c1, c2, c3)

            cl = lax.fori_loop(0, NV // 4, cnt_body,
                               (zeros16i, zeros16i, zeros16i, zeros16i))
            cnt = _lanesum(cl[0] + cl[1] + cl[2] + cl[3])
            hit = cnt == k
            sti_ref[pl.ds(0, 16)] = jnp.where(cnt >= k, trial, prefix)
            sti_ref[pl.ds(16, 16)] = jnp.where(hit, 1, 0)
            stf_ref[...] = jnp.where(hit, trialf, inf16)
            return jnp.int32(0)

        done = sti_ref[pl.ds(16, 16)]
        return lax.cond(done[0] == 1, lambda _: jnp.int32(0), active, 0)

    lax.fori_loop(1, 32, round_body, jnp.int32(0))
    done = sti_ref[pl.ds(16, 16)]
    tvec = jnp.where(done == 1, stf_ref[...],
                     lax.bitcast_convert_type(sti_ref[pl.ds(0, 16)], jnp.float32))

    def gt_body(i, c):
        mv = mv_ref[pl.ds(i * 16, 16)]
        return c + jnp.where(mv > tvec, 1, 0)

    cnt_gt = _lanesum(lax.fori_loop(0, NV, gt_body, zeros16i))
    m_extra = jnp.where(k > 0, k - cnt_gt, 0)
    tvec = jnp.where(k > 0, tvec, jnp.float32(jnp.inf))

    def fin_body(i, carry):
        cnt_eq, acc = carry
        mv = mv_ref[pl.ds(i * 16, 16)]
        cev = ce_ref[pl.ds(i * 16, 16)]
        eq = mv == tvec
        eqi = jnp.where(eq, 1, 0)
        c = _cumsum16(eqi)
        sel = eq & ((cnt_eq + c) <= m_extra)
        mask = (mv == 0.0) | (mv > tvec) | sel
        acc = acc + jnp.where(mask, cev, 0.0)
        return cnt_eq + _lanesum(eqi), acc

    _, acc = lax.fori_loop(0, NV, fin_body,
                           (zeros16i, jnp.zeros((16,), jnp.float32)))
    acc_ref[...] = acc
    pltpu.sync_copy(acc_ref, out_hbm.at[wid])


def _sc_mine(mval, ce, scal):
    mesh = plsc.VectorSubcoreMesh(core_axis_name="c", subcore_axis_name="s")
    f = pl.kernel(
        _sc_mine_kernel,
        mesh=mesh,
        out_type=jax.ShapeDtypeStruct((B, 16), jnp.float32),
        scratch_types=[
            pltpu.VMEM((P,), jnp.float32),
            pltpu.VMEM((P,), jnp.float32),
            pltpu.VMEM((128,), jnp.float32),
            pltpu.VMEM((16,), jnp.float32),
            pltpu.VMEM((32,), jnp.int32),
            pltpu.VMEM((16,), jnp.float32),
        ],
    )
    return f(mval, ce, scal)


def _sel_one(mv, cev, kk):
    bits = lax.bitcast_convert_type(mv, jnp.int32)
    prefix = jnp.int32(0)
    kp = kk.astype(jnp.int32)
    shifts_c = [31, 24, 16, 8]
    shifts_d = [24, 16, 8, 0]
    for r in range(4):
        cand = lax.shift_right_logical(bits, shifts_c[r]) == prefix
        digit = lax.shift_right_logical(bits, shifts_d[r]) & 255
        hist = jnp.zeros((256,), jnp.int32).at[digit].add(cand.astype(jnp.int32))
        suf = jnp.cumsum(hist[::-1])[::-1]
        ok = suf >= kp
        dstar = jnp.max(jnp.where(ok, jnp.arange(256), -1))
        cnt_gt = suf[jnp.minimum(dstar + 1, 255)] * (dstar < 255)
        kp = kp - cnt_gt
        prefix = (prefix << 8) | dstar
    t = lax.bitcast_convert_type(prefix, jnp.float32)
    m_extra = kp
    t = jnp.where(kk > 0, t, jnp.float32(jnp.inf))
    m_extra = jnp.where(kk > 0, m_extra, 0)
    eq = mv == t
    csum_eq = jnp.cumsum(eq.astype(jnp.int32))
    sel_eq = eq & (csum_eq <= m_extra)
    mask = (mv == 0.0) | (mv > t) | sel_eq
    return jnp.sum(jnp.where(mask, cev, 0.0))


def kernel(predictions, targets, priors):
    pt = priors.T.reshape(4, ROWS, LANES)
    confT = jnp.transpose(predictions[:, :, 4:], (0, 2, 1)).reshape(B, C, ROWS, LANES)
    locT = jnp.transpose(predictions[:, :, :4], (0, 2, 1)).reshape(B, 4, ROWS, LANES)

    bto, bti, bpi = _tc_match(pt, targets)
    ce, mval, scal = _tc_loss(confT, locT, pt, targets, bto, bti, bpi)

    npos = scal[:, 0, 0]
    csum = _sc_mine(mval.reshape(B, P), ce.reshape(B, P), scal.reshape(B, 128))

    bp = jnp.sum(npos)
    loss_l = jnp.sum(scal[:, 0, 1]) / bp
    loss_c = jnp.sum(csum) / bp
    return loss_l, loss_c


# speedup vs baseline: 1.0227x; 1.0027x over previous
import jax
import jax.numpy as jnp
from jax import lax
from jax.experimental import pallas as pl
from jax.experimental.pallas import tpu as pltpu
from jax.experimental.pallas import tpu_sc as plsc

C = 21
B = 32
P = 24576
O = 20
ROWS = 96
LANES = 256


def _iota2(shape):
    ri = lax.broadcasted_iota(jnp.int32, shape, 0)
    li = lax.broadcasted_iota(jnp.int32, shape, 1)
    return ri, li


def _match_kernel(pt_ref, tgt_ref, bto_ref, bti_ref, bpi_ref):
    pt = pt_ref[...]
    pcx, pcy, pw, ph = pt[0], pt[1], pt[2], pt[3]
    px1 = pcx - pw * 0.5
    py1 = pcy - ph * 0.5
    px2 = pcx + pw * 0.5
    py2 = pcy + ph * 0.5
    pa = (px2 - px1) * (py2 - py1)
    ri, li = _iota2((ROWS, LANES))
    gidx = ri * LANES + li

    b_ov = jnp.full((ROWS, LANES), -1.0, jnp.float32)
    b_ix = jnp.zeros((ROWS, LANES), jnp.int32)
    rm_list, cm_list = [], []
    for j in range(O):
        tx1 = tgt_ref[0, j, 0]
        ty1 = tgt_ref[0, j, 1]
        tx2 = tgt_ref[0, j, 2]
        ty2 = tgt_ref[0, j, 3]
        ta = (tx2 - tx1) * (ty2 - ty1)
        ix = jnp.maximum(jnp.minimum(tx2, px2) - jnp.maximum(tx1, px1), 0.0)
        iy = jnp.maximum(jnp.minimum(ty2, py2) - jnp.maximum(ty1, py1), 0.0)
        inter = ix * iy
        ov = inter / (ta + pa - inter)
        upd = ov > b_ov
        b_ov = jnp.where(upd, ov, b_ov)
        b_ix = jnp.where(upd, j, b_ix)
        rm = jnp.max(ov, axis=0)
        cm = jnp.min(jnp.where(ov == rm[None, :], gidx, P), axis=0)
        rm_list.append(rm)
        cm_list.append(cm)

    RM = jnp.stack(rm_list)
    CM = jnp.stack(cm_list)
    M = jnp.max(RM, axis=1, keepdims=True)
    idxv = jnp.min(jnp.where(RM == M, CM, P), axis=1)
    bpi_ref[...] = jnp.concatenate(
        [idxv, jnp.full((32 - O,), P, jnp.int32)]).reshape(1, 1, 32)

    bto_ref[0] = b_ov
    bti_ref[0] = b_ix


def _loss_kernel(confT_ref, locT_ref, pt_ref, tgt_ref, bto_ref, bti_ref,
                 bpi_ref, ce_ref, mval_ref, scal_ref):
    ri, li = _iota2((ROWS, LANES))
    gidx = ri * LANES + li

    bto = bto_ref[0]
    bti = bti_ref[0]
    sl32 = lax.broadcasted_iota(jnp.int32, (1, 1, 32), 2)
    bv = bpi_ref[...]
    for j in range(O):
        mfm = gidx == jnp.sum(jnp.where(sl32 == j, bv, 0))
        bto = jnp.where(mfm, 2.0, bto)
        bti = jnp.where(mfm, j, bti)
    pos = bto >= 0.5
    posf = pos.astype(jnp.float32)

    lab = jnp.zeros((ROWS, LANES), jnp.float32)
    mx1 = jnp.zeros((ROWS, LANES), jnp.float32)
    my1 = jnp.zeros((ROWS, LANES), jnp.float32)
    mx2 = jnp.zeros((ROWS, LANES), jnp.float32)
    my2 = jnp.zeros((ROWS, LANES), jnp.float32)
    for j in range(O):
        mj = bti == j
        lab = jnp.where(mj, tgt_ref[0, j, 4], lab)
        mx1 = jnp.where(mj, tgt_ref[0, j, 0], mx1)
        my1 = jnp.where(mj, tgt_ref[0, j, 1], my1)
        mx2 = jnp.where(mj, tgt_ref[0, j, 2], mx2)
        my2 = jnp.where(mj, tgt_ref[0, j, 3], my2)
    cj = jnp.where(pos, lab.astype(jnp.int32) + 1, 0)

    x = confT_ref[0]
    m = jnp.max(x, axis=0)
    e = jnp.exp(x - m[None])
    ssum = jnp.sum(e, axis=0)
    lse = jnp.log(ssum) + m
    ci = lax.broadcasted_iota(jnp.int32, (C, ROWS, LANES), 0)
    tgt = jnp.sum(jnp.where(ci == cj[None], x, 0.0), axis=0)
    ce = lse - tgt
    mval = jnp.where(pos, 0.0, ce)
    ce_ref[0] = ce
    mval_ref[0] = mval

    pt = pt_ref[...]
    pcx, pcy, pw, ph = pt[0], pt[1], pt[2], pt[3]
    g_cx = ((mx1 + mx2) * 0.5 - pcx) / (0.1 * pw)
    g_cy = ((my1 + my2) * 0.5 - pcy) / (0.1 * ph)
    g_w = jnp.log(jnp.maximum((mx2 - mx1) / pw, 1e-8)) / jnp.float32(0.2)
    g_h = jnp.log(jnp.maximum((my2 - my1) / ph, 1e-8)) / jnp.float32(0.2)
    lp = locT_ref[0]

    def sl1(d):
        ad = jnp.abs(d)
        return jnp.where(ad < 1.0, 0.5 * d * d, ad - 0.5)

    lsum = jnp.sum((sl1(lp[0] - g_cx) + sl1(lp[1] - g_cy)
                    + sl1(lp[2] - g_w) + sl1(lp[3] - g_h)) * posf)
    npc = jnp.sum(posf)

    slc = lax.broadcasted_iota(jnp.int32, (1, 1, 128), 2)
    scal_ref[...] = jnp.where(slc == 0, npc, 0.0) + jnp.where(slc == 1, lsum, 0.0)


def _tc_match(pt, targets):
    return pl.pallas_call(
        _match_kernel,
        grid=(B,),
        in_specs=[
            pl.BlockSpec((4, ROWS, LANES), lambda b: (0, 0, 0)),
            pl.BlockSpec((1, O, 5), lambda b: (b, 0, 0)),
        ],
        out_specs=[
            pl.BlockSpec((1, ROWS, LANES), lambda b: (b, 0, 0)),
            pl.BlockSpec((1, ROWS, LANES), lambda b: (b, 0, 0)),
            pl.BlockSpec((1, 1, 32), lambda b: (b, 0, 0)),
        ],
        out_shape=[
            jax.ShapeDtypeStruct((B, ROWS, LANES), jnp.float32),
            jax.ShapeDtypeStruct((B, ROWS, LANES), jnp.int32),
            jax.ShapeDtypeStruct((B, 1, 32), jnp.int32),
        ],
        compiler_params=pltpu.CompilerParams(
            dimension_semantics=("arbitrary",)),
    )(pt, targets)


def _tc_loss(confT, locT, pt, targets, bto, bti, bpi):
    return pl.pallas_call(
        _loss_kernel,
        grid=(B,),
        in_specs=[
            pl.BlockSpec((1, C, ROWS, LANES), lambda b: (b, 0, 0, 0)),
            pl.BlockSpec((1, 4, ROWS, LANES), lambda b: (b, 0, 0, 0)),
            pl.BlockSpec((4, ROWS, LANES), lambda b: (0, 0, 0)),
            pl.BlockSpec((1, O, 5), lambda b: (b, 0, 0)),
            pl.BlockSpec((1, ROWS, LANES), lambda b: (b, 0, 0)),
            pl.BlockSpec((1, ROWS, LANES), lambda b: (b, 0, 0)),
            pl.BlockSpec((1, 1, 32), lambda b: (b, 0, 0)),
        ],
        out_specs=[
            pl.BlockSpec((1, ROWS, LANES), lambda b: (b, 0, 0)),
            pl.BlockSpec((1, ROWS, LANES), lambda b: (b, 0, 0)),
            pl.BlockSpec((1, 1, 128), lambda b: (b, 0, 0)),
        ],
        out_shape=[
            jax.ShapeDtypeStruct((B, ROWS, LANES), jnp.float32),
            jax.ShapeDtypeStruct((B, ROWS, LANES), jnp.float32),
            jax.ShapeDtypeStruct((B, 1, 128), jnp.float32),
        ],
        compiler_params=pltpu.CompilerParams(
            dimension_semantics=("arbitrary",)),
    )(confT, locT, pt, targets, bto, bti, bpi)


NV = P // 16


def _take16(x, idx):
    return x.at[idx].get(mode="promise_in_bounds")


def _lanesum(x):
    i16 = lax.iota(jnp.int32, 16)
    for d in (8, 4, 2, 1):
        x = x + _take16(x, i16 ^ d)
    return x


def _cumsum16(x):
    i16 = lax.iota(jnp.int32, 16)
    for d in (1, 2, 4, 8):
        sh = _take16(x, jnp.maximum(i16 - d, 0))
        x = x + jnp.where(i16 >= d, sh, 0)
    return x


def _sc_mine_kernel(mval_hbm, ce_hbm, scal_hbm, out_hbm,
                    mv_ref, ce_ref, sc_ref, acc_ref):
    wid = lax.axis_index("s") * 2 + lax.axis_index("c")
    pltpu.sync_copy(mval_hbm.at[wid], mv_ref)
    pltpu.sync_copy(ce_hbm.at[wid], ce_ref)
    pltpu.sync_copy(scal_hbm.at[wid], sc_ref)

    i16 = lax.iota(jnp.int32, 16)
    zeros16i = jnp.zeros((16,), jnp.int32)

    head = sc_ref[pl.ds(0, 16)].astype(jnp.int32)
    npos = _lanesum(jnp.where(i16 == 0, head, 0))
    k = jnp.minimum(3 * npos, P - 1)

    def round_body(r, prefix):
        trial = prefix | lax.shift_left(jnp.int32(1), 31 - r)
        trialf = lax.bitcast_convert_type(trial, jnp.float32)

        def cnt_body(i, cs):
            c0, c1, c2, c3 = cs
            b = i * 64
            c0 = c0 + jnp.where(mv_ref[pl.ds(b, 16)] >= trialf, 1, 0)
            c1 = c1 + jnp.where(mv_ref[pl.ds(b + 16, 16)] >= trialf, 1, 0)
            c2 = c2 + jnp.where(mv_ref[pl.ds(b + 32, 16)] >= trialf, 1, 0)
            c3 = c3 + jnp.where(mv_ref[pl.ds(b + 48, 16)] >= trialf, 1, 0)
            return (c0, c1, c2, c3)

        cl = lax.fori_loop(0, NV // 4, cnt_body,
                           (zeros16i, zeros16i, zeros16i, zeros16i))
        cnt = _lanesum(cl[0] + cl[1] + cl[2] + cl[3])
        return jnp.where(cnt >= k, trial, prefix)

    prefix = lax.fori_loop(1, 32, round_body, zeros16i)
    tvec = lax.bitcast_convert_type(prefix, jnp.float32)

    def gt_body(i, c):
        mv = mv_ref[pl.ds(i * 16, 16)]
        return c + jnp.where(mv > tvec, 1, 0)

    cnt_gt = _lanesum(lax.fori_loop(0, NV, gt_body, zeros16i))
    m_extra = jnp.where(k > 0, k - cnt_gt, 0)
    tvec = jnp.where(k > 0, tvec, jnp.float32(jnp.inf))

    def fin_body(i, carry):
        cnt_eq, acc = carry
        mv = mv_ref[pl.ds(i * 16, 16)]
        cev = ce_ref[pl.ds(i * 16, 16)]
        eq = mv == tvec
        eqi = jnp.where(eq, 1, 0)
        c = _cumsum16(eqi)
        sel = eq & ((cnt_eq + c) <= m_extra)
        mask = (mv == 0.0) | (mv > tvec) | sel
        acc = acc + jnp.where(mask, cev, 0.0)
        return cnt_eq + _lanesum(eqi), acc

    _, acc = lax.fori_loop(0, NV, fin_body,
                           (zeros16i, jnp.zeros((16,), jnp.float32)))
    acc_ref[...] = acc
    pltpu.sync_copy(acc_ref, out_hbm.at[wid])


def _sc_mine(mval, ce, scal):
    mesh = plsc.VectorSubcoreMesh(core_axis_name="c", subcore_axis_name="s")
    f = pl.kernel(
        _sc_mine_kernel,
        mesh=mesh,
        out_type=jax.ShapeDtypeStruct((B, 16), jnp.float32),
        scratch_types=[
            pltpu.VMEM((P,), jnp.float32),
            pltpu.VMEM((P,), jnp.float32),
            pltpu.VMEM((128,), jnp.float32),
            pltpu.VMEM((16,), jnp.float32),
        ],
    )
    return f(mval, ce, scal)


def kernel(predictions, targets, priors):
    pt = priors.T.reshape(4, ROWS, LANES)
    confT = jnp.transpose(predictions[:, :, 4:], (0, 2, 1)).reshape(B, C, ROWS, LANES)
    locT = jnp.transpose(predictions[:, :, :4], (0, 2, 1)).reshape(B, 4, ROWS, LANES)

    bto, bti, bpi = _tc_match(pt, targets)
    ce, mval, scal = _tc_loss(confT, locT, pt, targets, bto, bti, bpi)

    npos = scal[:, 0, 0]
    csum = _sc_mine(mval.reshape(B, P), ce.reshape(B, P), scal.reshape(B, 128))

    bp = jnp.sum(npos)
    loss_l = jnp.sum(scal[:, 0, 1]) / bp
    loss_c = jnp.sum(csum) / bp
    return loss_l, loss_c


# speedup vs baseline: 1.2347x; 1.2072x over previous
import jax
import jax.numpy as jnp
from jax import lax
from jax.experimental import pallas as pl
from jax.experimental.pallas import tpu as pltpu
from jax.experimental.pallas import tpu_sc as plsc

C = 21
B = 32
P = 24576
O = 20
ROWS = 96
LANES = 256


def _iota2(shape):
    ri = lax.broadcasted_iota(jnp.int32, shape, 0)
    li = lax.broadcasted_iota(jnp.int32, shape, 1)
    return ri, li


def _match_kernel(pt_ref, tgt_ref, bto_ref, bti_ref, bpi_ref):
    pt = pt_ref[...]
    pcx, pcy, pw, ph = pt[0], pt[1], pt[2], pt[3]
    px1 = pcx - pw * 0.5
    py1 = pcy - ph * 0.5
    px2 = pcx + pw * 0.5
    py2 = pcy + ph * 0.5
    pa = (px2 - px1) * (py2 - py1)
    ri, li = _iota2((ROWS, LANES))
    gidx = ri * LANES + li

    b_ov = jnp.full((ROWS, LANES), -1.0, jnp.float32)
    b_ix = jnp.zeros((ROWS, LANES), jnp.int32)
    rm_list, cm_list = [], []
    for j in range(O):
        tx1 = tgt_ref[0, j, 0]
        ty1 = tgt_ref[0, j, 1]
        tx2 = tgt_ref[0, j, 2]
        ty2 = tgt_ref[0, j, 3]
        ta = (tx2 - tx1) * (ty2 - ty1)
        ix = jnp.maximum(jnp.minimum(tx2, px2) - jnp.maximum(tx1, px1), 0.0)
        iy = jnp.maximum(jnp.minimum(ty2, py2) - jnp.maximum(ty1, py1), 0.0)
        inter = ix * iy
        ov = inter / (ta + pa - inter)
        upd = ov > b_ov
        b_ov = jnp.where(upd, ov, b_ov)
        b_ix = jnp.where(upd, j, b_ix)
        rm = jnp.max(ov, axis=0)
        cm = jnp.min(jnp.where(ov == rm[None, :], gidx, P), axis=0)
        rm_list.append(rm)
        cm_list.append(cm)

    RM = jnp.stack(rm_list)
    CM = jnp.stack(cm_list)
    M = jnp.max(RM, axis=1, keepdims=True)
    idxv = jnp.min(jnp.where(RM == M, CM, P), axis=1)
    bpi_ref[...] = jnp.concatenate(
        [idxv, jnp.full((32 - O,), P, jnp.int32)]).reshape(1, 1, 32)

    bto_ref[0] = b_ov
    bti_ref[0] = b_ix


def _loss_kernel(xT_ref, pt_ref, tgt_ref, bto_ref, bti_ref,
                 bpi_ref, ce_ref, mval_ref, scal_ref):
    ri, li = _iota2((ROWS, LANES))
    gidx = ri * LANES + li

    bto = bto_ref[0]
    bti = bti_ref[0]
    sl32 = lax.broadcasted_iota(jnp.int32, (1, 1, 32), 2)
    bv = bpi_ref[...]
    for j in range(O):
        mfm = gidx == jnp.sum(jnp.where(sl32 == j, bv, 0))
        bto = jnp.where(mfm, 2.0, bto)
        bti = jnp.where(mfm, j, bti)
    pos = bto >= 0.5
    posf = pos.astype(jnp.float32)

    lab = jnp.zeros((ROWS, LANES), jnp.float32)
    mx1 = jnp.zeros((ROWS, LANES), jnp.float32)
    my1 = jnp.zeros((ROWS, LANES), jnp.float32)
    mx2 = jnp.zeros((ROWS, LANES), jnp.float32)
    my2 = jnp.zeros((ROWS, LANES), jnp.float32)
    for j in range(O):
        mj = bti == j
        lab = jnp.where(mj, tgt_ref[0, j, 4], lab)
        mx1 = jnp.where(mj, tgt_ref[0, j, 0], mx1)
        my1 = jnp.where(mj, tgt_ref[0, j, 1], my1)
        mx2 = jnp.where(mj, tgt_ref[0, j, 2], mx2)
        my2 = jnp.where(mj, tgt_ref[0, j, 3], my2)
    cj = jnp.where(pos, lab.astype(jnp.int32) + 1, 0)

    x_all = xT_ref[0]
    x = x_all[4:25]
    m = jnp.max(x, axis=0)
    e = jnp.exp(x - m[None])
    ssum = jnp.sum(e, axis=0)
    lse = jnp.log(ssum) + m
    ci = lax.broadcasted_iota(jnp.int32, (C, ROWS, LANES), 0)
    tgt = jnp.sum(jnp.where(ci == cj[None], x, 0.0), axis=0)
    ce = lse - tgt
    mval = jnp.where(pos, 0.0, ce)
    ce_ref[0] = ce
    mval_ref[0] = mval

    pt = pt_ref[...]
    pcx, pcy, pw, ph = pt[0], pt[1], pt[2], pt[3]
    g_cx = ((mx1 + mx2) * 0.5 - pcx) / (0.1 * pw)
    g_cy = ((my1 + my2) * 0.5 - pcy) / (0.1 * ph)
    g_w = jnp.log(jnp.maximum((mx2 - mx1) / pw, 1e-8)) / jnp.float32(0.2)
    g_h = jnp.log(jnp.maximum((my2 - my1) / ph, 1e-8)) / jnp.float32(0.2)
    lp = x_all[0:4]

    def sl1(d):
        ad = jnp.abs(d)
        return jnp.where(ad < 1.0, 0.5 * d * d, ad - 0.5)

    lsum = jnp.sum((sl1(lp[0] - g_cx) + sl1(lp[1] - g_cy)
                    + sl1(lp[2] - g_w) + sl1(lp[3] - g_h)) * posf)
    npc = jnp.sum(posf)

    slc = lax.broadcasted_iota(jnp.int32, (1, 1, 128), 2)
    scal_ref[...] = jnp.where(slc == 0, npc, 0.0) + jnp.where(slc == 1, lsum, 0.0)


def _tc_match(pt, targets):
    return pl.pallas_call(
        _match_kernel,
        grid=(B,),
        in_specs=[
            pl.BlockSpec((4, ROWS, LANES), lambda b: (0, 0, 0)),
            pl.BlockSpec((1, O, 5), lambda b: (b, 0, 0)),
        ],
        out_specs=[
            pl.BlockSpec((1, ROWS, LANES), lambda b: (b, 0, 0)),
            pl.BlockSpec((1, ROWS, LANES), lambda b: (b, 0, 0)),
            pl.BlockSpec((1, 1, 32), lambda b: (b, 0, 0)),
        ],
        out_shape=[
            jax.ShapeDtypeStruct((B, ROWS, LANES), jnp.float32),
            jax.ShapeDtypeStruct((B, ROWS, LANES), jnp.int32),
            jax.ShapeDtypeStruct((B, 1, 32), jnp.int32),
        ],
        compiler_params=pltpu.CompilerParams(
            dimension_semantics=("arbitrary",)),
    )(pt, targets)


def _tc_loss(xT, pt, targets, bto, bti, bpi):
    return pl.pallas_call(
        _loss_kernel,
        grid=(B,),
        in_specs=[
            pl.BlockSpec((1, 25, ROWS, LANES), lambda b: (b, 0, 0, 0)),
            pl.BlockSpec((4, ROWS, LANES), lambda b: (0, 0, 0)),
            pl.BlockSpec((1, O, 5), lambda b: (b, 0, 0)),
            pl.BlockSpec((1, ROWS, LANES), lambda b: (b, 0, 0)),
            pl.BlockSpec((1, ROWS, LANES), lambda b: (b, 0, 0)),
            pl.BlockSpec((1, 1, 32), lambda b: (b, 0, 0)),
        ],
        out_specs=[
            pl.BlockSpec((1, ROWS, LANES), lambda b: (b, 0, 0)),
            pl.BlockSpec((1, ROWS, LANES), lambda b: (b, 0, 0)),
            pl.BlockSpec((1, 1, 128), lambda b: (b, 0, 0)),
        ],
        out_shape=[
            jax.ShapeDtypeStruct((B, ROWS, LANES), jnp.float32),
            jax.ShapeDtypeStruct((B, ROWS, LANES), jnp.float32),
            jax.ShapeDtypeStruct((B, 1, 128), jnp.float32),
        ],
        compiler_params=pltpu.CompilerParams(
            dimension_semantics=("arbitrary",)),
    )(xT, pt, targets, bto, bti, bpi)


NV = P // 16


def _take16(x, idx):
    return x.at[idx].get(mode="promise_in_bounds")


def _lanesum(x):
    i16 = lax.iota(jnp.int32, 16)
    for d in (8, 4, 2, 1):
        x = x + _take16(x, i16 ^ d)
    return x


def _cumsum16(x):
    i16 = lax.iota(jnp.int32, 16)
    for d in (1, 2, 4, 8):
        sh = _take16(x, jnp.maximum(i16 - d, 0))
        x = x + jnp.where(i16 >= d, sh, 0)
    return x


def _sc_mine_kernel(mval_hbm, ce_hbm, scal_hbm, out_hbm,
                    mv_ref, ce_ref, sc_ref, acc_ref):
    wid = lax.axis_index("s") * 2 + lax.axis_index("c")
    pltpu.sync_copy(mval_hbm.at[wid], mv_ref)
    pltpu.sync_copy(ce_hbm.at[wid], ce_ref)
    pltpu.sync_copy(scal_hbm.at[wid], sc_ref)

    i16 = lax.iota(jnp.int32, 16)
    zeros16i = jnp.zeros((16,), jnp.int32)

    head = sc_ref[pl.ds(0, 16)].astype(jnp.int32)
    npos = _lanesum(jnp.where(i16 == 0, head, 0))
    k = jnp.minimum(3 * npos, P - 1)

    def round_body(r, prefix):
        trial = prefix | lax.shift_left(jnp.int32(1), 31 - r)
        trialf = lax.bitcast_convert_type(trial, jnp.float32)

        def cnt_body(i, cs):
            c0, c1, c2, c3 = cs
            b = i * 64
            c0 = c0 + jnp.where(mv_ref[pl.ds(b, 16)] >= trialf, 1, 0)
            c1 = c1 + jnp.where(mv_ref[pl.ds(b + 16, 16)] >= trialf, 1, 0)
            c2 = c2 + jnp.where(mv_ref[pl.ds(b + 32, 16)] >= trialf, 1, 0)
            c3 = c3 + jnp.where(mv_ref[pl.ds(b + 48, 16)] >= trialf, 1, 0)
            return (c0, c1, c2, c3)

        cl = lax.fori_loop(0, NV // 4, cnt_body,
                           (zeros16i, zeros16i, zeros16i, zeros16i))
        cnt = _lanesum(cl[0] + cl[1] + cl[2] + cl[3])
        return jnp.where(cnt >= k, trial, prefix)

    prefix = lax.fori_loop(1, 32, round_body, zeros16i)
    tvec = lax.bitcast_convert_type(prefix, jnp.float32)

    def gt_body(i, c):
        mv = mv_ref[pl.ds(i * 16, 16)]
        return c + jnp.where(mv > tvec, 1, 0)

    cnt_gt = _lanesum(lax.fori_loop(0, NV, gt_body, zeros16i))
    m_extra = jnp.where(k > 0, k - cnt_gt, 0)
    tvec = jnp.where(k > 0, tvec, jnp.float32(jnp.inf))

    def fin_body(i, carry):
        cnt_eq, acc = carry
        mv = mv_ref[pl.ds(i * 16, 16)]
        cev = ce_ref[pl.ds(i * 16, 16)]
        eq = mv == tvec
        eqi = jnp.where(eq, 1, 0)
        c = _cumsum16(eqi)
        sel = eq & ((cnt_eq + c) <= m_extra)
        mask = (mv == 0.0) | (mv > tvec) | sel
        acc = acc + jnp.where(mask, cev, 0.0)
        return cnt_eq + _lanesum(eqi), acc

    _, acc = lax.fori_loop(0, NV, fin_body,
                           (zeros16i, jnp.zeros((16,), jnp.float32)))
    acc_ref[...] = acc
    pltpu.sync_copy(acc_ref, out_hbm.at[wid])


def _sc_mine(mval, ce, scal):
    mesh = plsc.VectorSubcoreMesh(core_axis_name="c", subcore_axis_name="s")
    f = pl.kernel(
        _sc_mine_kernel,
        mesh=mesh,
        out_type=jax.ShapeDtypeStruct((B, 16), jnp.float32),
        scratch_types=[
            pltpu.VMEM((P,), jnp.float32),
            pltpu.VMEM((P,), jnp.float32),
            pltpu.VMEM((128,), jnp.float32),
            pltpu.VMEM((16,), jnp.float32),
        ],
    )
    return f(mval, ce, scal)


def kernel(predictions, targets, priors):
    pt = priors.T.reshape(4, ROWS, LANES)
    xT = jnp.transpose(predictions, (0, 2, 1)).reshape(B, 25, ROWS, LANES)

    bto, bti, bpi = _tc_match(pt, targets)
    ce, mval, scal = _tc_loss(xT, pt, targets, bto, bti, bpi)

    npos = scal[:, 0, 0]
    csum = _sc_mine(mval.reshape(B, P), ce.reshape(B, P), scal.reshape(B, 128))

    bp = jnp.sum(npos)
    loss_l = jnp.sum(scal[:, 0, 1]) / bp
    loss_c = jnp.sum(csum) / bp
    return loss_l, loss_c
